# baseline (speedup 1.0000x reference)
import functools

import jax
import jax.numpy as jnp
from jax import lax
from jax.experimental import pallas as pl
from jax.experimental.pallas import tpu as pltpu
from jax.experimental.pallas import tpu_sc as plsc

_BETA = 0.25

_N = 8192
_K = 8192
_D = 256

_TNA = 2048
_TKA = 2048
_GNA = _N // _TNA
_GKA = _K // _TKA
_TNB = 512
_TKB = 8192
_GNB = _N // _TNB
_GKB = _K // _TKB


def _dist_body(z_ref, cb_ref, dist_ref, idx_ref, loss_ref, rmin_ref, ridx_ref,
               acc_ref, zn_ref):
    n = pl.program_id(0)
    k = pl.program_id(1)
    zt = z_ref[...]
    ct = cb_ref[...]

    @pl.when(k == 0)
    def _():
        zn_ref[...] = jnp.sum(zt * zt, axis=1, keepdims=True)

    zn = zn_ref[...]
    cn = jnp.sum(ct * ct, axis=1)
    mm = lax.dot_general(zt * (-2.0), ct, (((1,), (1,)), ((), ())),
                         preferred_element_type=jnp.float32)
    dist = zn + cn[None, :] + mm
    dist_ref[...] = dist

    lmin = jnp.min(dist, axis=1, keepdims=True)
    iota = lax.broadcasted_iota(jnp.int32, (1, _TKA), 1).astype(jnp.float32)
    cand = jnp.min(jnp.where(dist == lmin, iota, jnp.float32(_TKA)),
                   axis=1, keepdims=True)
    larg = cand.astype(jnp.int32) + k * _TKA

    @pl.when(k == 0)
    def _():
        rmin_ref[...] = lmin
        ridx_ref[...] = larg

    @pl.when(k > 0)
    def _():
        pm = rmin_ref[...]
        upd = lmin < pm
        rmin_ref[...] = jnp.where(upd, lmin, pm)
        ridx_ref[...] = jnp.where(upd, larg, ridx_ref[...])

    @pl.when(k == _GKA - 1)
    def _():
        idx_ref[0, :, :] = ridx_ref[...]
        s = jnp.sum(rmin_ref[...])

        @pl.when(n == 0)
        def _():
            acc_ref[0, 0] = s

        @pl.when(n > 0)
        def _():
            acc_ref[0, 0] += s

        @pl.when(n == _GNA - 1)
        def _():
            loss_ref[0, 0] = acc_ref[0, 0] * ((1.0 + _BETA) / (_N * _D))


def _distances_argmin(z_flat, codebook):
    return pl.pallas_call(
        _dist_body,
        grid=(_GNA, _GKA),
        in_specs=[
            pl.BlockSpec((_TNA, _D), lambda n, k: (n, 0)),
            pl.BlockSpec((_TKA, _D), lambda n, k: (k, 0)),
        ],
        out_specs=[
            pl.BlockSpec((_TNA, _TKA), lambda n, k: (n, k)),
            pl.BlockSpec((1, _TNA, 1), lambda n, k: (n, 0, 0)),
            pl.BlockSpec(memory_space=pltpu.SMEM, block_shape=(1, 1),
                         index_map=lambda n, k: (0, 0)),
        ],
        out_shape=[
            jax.ShapeDtypeStruct((_N, _K), jnp.float32),
            jax.ShapeDtypeStruct((_GNA, _TNA, 1), jnp.int32),
            jax.ShapeDtypeStruct((1, 1), jnp.float32),
        ],
        scratch_shapes=[
            pltpu.VMEM((_TNA, 1), jnp.float32),
            pltpu.VMEM((_TNA, 1), jnp.int32),
            pltpu.SMEM((1, 1), jnp.float32),
            pltpu.VMEM((_TNA, 1), jnp.float32),
        ],
    )(z_flat, codebook)


def _onehot_body(idx_ref, enc_ref, perp_ref, counts_ref):
    n = pl.program_id(0)
    k = pl.program_id(1)
    idxv = idx_ref[0, :, :]
    col = lax.broadcasted_iota(jnp.int32, (1, _TKB), 1) + k * _TKB
    enc = jnp.where(col == idxv, 1.0, 0.0)
    enc_ref[...] = enc
    cs = jnp.sum(enc, axis=0, keepdims=True)

    @pl.when(n == 0)
    def _():
        counts_ref[pl.ds(k, 1), :] = cs

    @pl.when(n > 0)
    def _():
        counts_ref[pl.ds(k, 1), :] += cs

    @pl.when((n == _GNB - 1) & (k == _GKB - 1))
    def _():
        p = counts_ref[...] * (1.0 / _N)
        ent = jnp.sum(p * jnp.log(p + 1e-10))
        perp_ref[0, 0] = jnp.exp(-ent)


def _onehot_perplexity(idx3):
    return pl.pallas_call(
        _onehot_body,
        grid=(_GNB, _GKB),
        in_specs=[
            pl.BlockSpec((1, _TNB, 1), lambda n, k: (n, 0, 0)),
        ],
        out_specs=[
            pl.BlockSpec((_TNB, _TKB), lambda n, k: (n, k)),
            pl.BlockSpec(memory_space=pltpu.SMEM, block_shape=(1, 1),
                         index_map=lambda n, k: (0, 0)),
        ],
        out_shape=[
            jax.ShapeDtypeStruct((_N, _K), jnp.float32),
            jax.ShapeDtypeStruct((1, 1), jnp.float32),
        ],
        scratch_shapes=[
            pltpu.VMEM((_GKB, _TKB), jnp.float32),
        ],
    )(idx3)


def _sc_gather(codebook, idx_flat):
    info = plsc.get_sparse_core_info()
    nc, ns = info.num_cores, info.num_subcores
    nw = nc * ns
    b_per_w = _N // nw
    mesh = plsc.VectorSubcoreMesh(core_axis_name="c", subcore_axis_name="s")

    half = b_per_w // 2

    @functools.partial(
        pl.kernel, mesh=mesh,
        out_type=jax.ShapeDtypeStruct((_N, _D), jnp.float32),
        scratch_types=[
            pltpu.VMEM((half,), jnp.int32),
            pltpu.VMEM((half,), jnp.int32),
            pltpu.VMEM((half, _D), jnp.float32),
            pltpu.VMEM((half, _D), jnp.float32),
            pltpu.SemaphoreType.DMA,
            pltpu.SemaphoreType.DMA,
        ],
    )
    def gather_kernel(cb_hbm, idx_hbm, out_hbm, idx0, idx1, rows0, rows1,
                      sem0, sem1):
        wid = lax.axis_index("s") * nc + lax.axis_index("c")
        base = wid * b_per_w
        pltpu.sync_copy(idx_hbm.at[pl.ds(base, half)], idx0)
        cp0 = pltpu.async_copy(cb_hbm.at[idx0], rows0, sem0)
        pltpu.sync_copy(idx_hbm.at[pl.ds(base + half, half)], idx1)
        cp1 = pltpu.async_copy(cb_hbm.at[idx1], rows1, sem1)
        cp0.wait()
        pltpu.sync_copy(rows0, out_hbm.at[pl.ds(base, half)])
        cp1.wait()
        pltpu.sync_copy(rows1, out_hbm.at[pl.ds(base + half, half)])

    return gather_kernel(codebook, idx_flat)


def kernel(z, codebook):
    B, D, H, W = z.shape
    z_flat = jnp.transpose(z, (0, 2, 3, 1)).reshape(_N, _D)
    distances, idx3, loss = _distances_argmin(z_flat, codebook)
    encoding_indices = idx3.reshape(_N)
    zq_flat = _sc_gather(codebook, encoding_indices)
    encodings, perp = _onehot_perplexity(idx3.reshape(_GNB, _TNB, 1))
    z_quantized = jnp.transpose(zq_flat.reshape(B, H, W, D), (0, 3, 1, 2))
    z_quantized = z + (z_quantized - z)
    loss = loss.reshape(())
    perplexity = perp.reshape(())
    return (z_quantized, loss, perplexity, encodings, encoding_indices,
            distances)


# speedup vs baseline: 1.0743x; 1.0743x over previous
import functools

import jax
import jax.numpy as jnp
from jax import lax
from jax.experimental import pallas as pl
from jax.experimental.pallas import tpu as pltpu
from jax.experimental.pallas import tpu_sc as plsc

_BETA = 0.25

_N = 8192
_K = 8192
_D = 256

_TNA = 2048
_TKA = 2048
_GNA = _N // _TNA
_GKA = _K // _TKA
_TNB = 1024
_TKB = 4096
_GNB = _N // _TNB
_GKB = _K // _TKB


def _dist_body(z_ref, cb_ref, dist_ref, idx_ref, loss_ref, rmin_ref, ridx_ref,
               acc_ref, zn_ref):
    n = pl.program_id(0)
    k = pl.program_id(1)
    zt = z_ref[...]
    ct = cb_ref[...]

    @pl.when(k == 0)
    def _():
        zn_ref[...] = jnp.sum(zt * zt, axis=1, keepdims=True)

    zn = zn_ref[...]
    cn = jnp.sum(ct * ct, axis=1)
    mm = lax.dot_general(zt * (-2.0), ct, (((1,), (1,)), ((), ())),
                         preferred_element_type=jnp.float32)
    dist = zn + cn[None, :] + mm
    dist_ref[...] = dist

    lmin = jnp.min(dist, axis=1, keepdims=True)
    iota = lax.broadcasted_iota(jnp.int32, (1, _TKA), 1).astype(jnp.float32)
    cand = jnp.min(jnp.where(dist == lmin, iota, jnp.float32(_TKA)),
                   axis=1, keepdims=True)
    larg = cand.astype(jnp.int32) + k * _TKA

    @pl.when(k == 0)
    def _():
        rmin_ref[...] = lmin
        ridx_ref[...] = larg

    @pl.when(k > 0)
    def _():
        pm = rmin_ref[...]
        upd = lmin < pm
        rmin_ref[...] = jnp.where(upd, lmin, pm)
        ridx_ref[...] = jnp.where(upd, larg, ridx_ref[...])

    @pl.when(k == _GKA - 1)
    def _():
        idx_ref[0, :, :] = ridx_ref[...]
        s = jnp.sum(rmin_ref[...])

        @pl.when(n == 0)
        def _():
            acc_ref[0, 0] = s

        @pl.when(n > 0)
        def _():
            acc_ref[0, 0] += s

        @pl.when(n == _GNA - 1)
        def _():
            loss_ref[0, 0] = acc_ref[0, 0] * ((1.0 + _BETA) / (_N * _D))


def _distances_argmin(z_flat, codebook):
    return pl.pallas_call(
        _dist_body,
        grid=(_GNA, _GKA),
        in_specs=[
            pl.BlockSpec((_TNA, _D), lambda n, k: (n, 0)),
            pl.BlockSpec((_TKA, _D), lambda n, k: (k, 0)),
        ],
        out_specs=[
            pl.BlockSpec((_TNA, _TKA), lambda n, k: (n, k)),
            pl.BlockSpec((1, _TNA, 1), lambda n, k: (n, 0, 0)),
            pl.BlockSpec(memory_space=pltpu.SMEM, block_shape=(1, 1),
                         index_map=lambda n, k: (0, 0)),
        ],
        out_shape=[
            jax.ShapeDtypeStruct((_N, _K), jnp.float32),
            jax.ShapeDtypeStruct((_GNA, _TNA, 1), jnp.int32),
            jax.ShapeDtypeStruct((1, 1), jnp.float32),
        ],
        scratch_shapes=[
            pltpu.VMEM((_TNA, 1), jnp.float32),
            pltpu.VMEM((_TNA, 1), jnp.int32),
            pltpu.SMEM((1, 1), jnp.float32),
            pltpu.VMEM((_TNA, 1), jnp.float32),
        ],
    )(z_flat, codebook)


def _onehot_body(idx_ref, enc_ref, perp_ref, counts_ref):
    n = pl.program_id(0)
    k = pl.program_id(1)
    idxv = idx_ref[0, :, :]
    col = lax.broadcasted_iota(jnp.int32, (1, _TKB), 1) + k * _TKB
    enc = jnp.where(col == idxv, 1.0, 0.0)
    enc_ref[...] = enc
    cs = jnp.sum(enc, axis=0, keepdims=True)

    @pl.when(n == 0)
    def _():
        counts_ref[pl.ds(k, 1), :] = cs

    @pl.when(n > 0)
    def _():
        counts_ref[pl.ds(k, 1), :] += cs

    @pl.when((n == _GNB - 1) & (k == _GKB - 1))
    def _():
        p = counts_ref[...] * (1.0 / _N)
        ent = jnp.sum(p * jnp.log(p + 1e-10))
        perp_ref[0, 0] = jnp.exp(-ent)


def _onehot_perplexity(idx3):
    return pl.pallas_call(
        _onehot_body,
        grid=(_GNB, _GKB),
        in_specs=[
            pl.BlockSpec((1, _TNB, 1), lambda n, k: (n, 0, 0)),
        ],
        out_specs=[
            pl.BlockSpec((_TNB, _TKB), lambda n, k: (n, k)),
            pl.BlockSpec(memory_space=pltpu.SMEM, block_shape=(1, 1),
                         index_map=lambda n, k: (0, 0)),
        ],
        out_shape=[
            jax.ShapeDtypeStruct((_N, _K), jnp.float32),
            jax.ShapeDtypeStruct((1, 1), jnp.float32),
        ],
        scratch_shapes=[
            pltpu.VMEM((_GKB, _TKB), jnp.float32),
        ],
    )(idx3)


def _sc_gather(codebook, idx_flat):
    info = plsc.get_sparse_core_info()
    nc, ns = info.num_cores, info.num_subcores
    nw = nc * ns
    b_per_w = _N // nw
    mesh = plsc.VectorSubcoreMesh(core_axis_name="c", subcore_axis_name="s")

    half = b_per_w // 2

    @functools.partial(
        pl.kernel, mesh=mesh,
        out_type=jax.ShapeDtypeStruct((_N, _D), jnp.float32),
        scratch_types=[
            pltpu.VMEM((half,), jnp.int32),
            pltpu.VMEM((half,), jnp.int32),
            pltpu.VMEM((half, _D), jnp.float32),
            pltpu.VMEM((half, _D), jnp.float32),
            pltpu.SemaphoreType.DMA,
            pltpu.SemaphoreType.DMA,
        ],
    )
    def gather_kernel(cb_hbm, idx_hbm, out_hbm, idx0, idx1, rows0, rows1,
                      sem0, sem1):
        wid = lax.axis_index("s") * nc + lax.axis_index("c")
        base = wid * b_per_w
        pltpu.sync_copy(idx_hbm.at[pl.ds(base, half)], idx0)
        cp0 = pltpu.async_copy(cb_hbm.at[idx0], rows0, sem0)
        pltpu.sync_copy(idx_hbm.at[pl.ds(base + half, half)], idx1)
        cp1 = pltpu.async_copy(cb_hbm.at[idx1], rows1, sem1)
        cp0.wait()
        pltpu.sync_copy(rows0, out_hbm.at[pl.ds(base, half)])
        cp1.wait()
        pltpu.sync_copy(rows1, out_hbm.at[pl.ds(base + half, half)])

    return gather_kernel(codebook, idx_flat)


def kernel(z, codebook):
    B, D, H, W = z.shape
    z_flat = jnp.transpose(z, (0, 2, 3, 1)).reshape(_N, _D)
    distances, idx3, loss = _distances_argmin(z_flat, codebook)
    encoding_indices = idx3.reshape(_N)
    zq_flat = _sc_gather(codebook, encoding_indices)
    encodings, perp = _onehot_perplexity(idx3.reshape(_GNB, _TNB, 1))
    z_quantized = jnp.transpose(zq_flat.reshape(B, H, W, D), (0, 3, 1, 2))
    loss = loss.reshape(())
    perplexity = perp.reshape(())
    return (z_quantized, loss, perplexity, encodings, encoding_indices,
            distances)


# speedup vs baseline: 1.0762x; 1.0018x over previous
import functools

import jax
import jax.numpy as jnp
from jax import lax
from jax.experimental import pallas as pl
from jax.experimental.pallas import tpu as pltpu
from jax.experimental.pallas import tpu_sc as plsc

_BETA = 0.25

_N = 8192
_K = 8192
_D = 256

_TNA = 2048
_TKA = 2048
_GNA = _N // _TNA
_GKA = _K // _TKA
_TNB = 1024
_TKB = 4096
_GNB = _N // _TNB
_GKB = _K // _TKB


def _dist_body(z_ref, cb_ref, dist_ref, idx_ref, loss_ref, rmin_ref, ridx_ref,
               acc_ref, zn_ref):
    n = pl.program_id(0)
    k = pl.program_id(1)
    zt = z_ref[...]
    ct = cb_ref[...]

    @pl.when(k == 0)
    def _():
        zn_ref[...] = jnp.sum(zt * zt, axis=1, keepdims=True)

    zn = zn_ref[...]
    cn = jnp.sum(ct * ct, axis=1)
    mm = lax.dot_general(zt * (-2.0), ct, (((1,), (1,)), ((), ())),
                         preferred_element_type=jnp.float32)
    dist = zn + cn[None, :] + mm
    dist_ref[...] = dist

    lmin = jnp.min(dist, axis=1, keepdims=True)
    iota = lax.broadcasted_iota(jnp.int32, (1, _TKA), 1).astype(jnp.float32)
    cand = jnp.min(jnp.where(dist == lmin, iota, jnp.float32(_TKA)),
                   axis=1, keepdims=True)
    larg = cand.astype(jnp.int32) + k * _TKA

    @pl.when(k == 0)
    def _():
        rmin_ref[...] = lmin
        ridx_ref[...] = larg

    @pl.when(k > 0)
    def _():
        pm = rmin_ref[...]
        upd = lmin < pm
        rmin_ref[...] = jnp.where(upd, lmin, pm)
        ridx_ref[...] = jnp.where(upd, larg, ridx_ref[...])

    @pl.when(k == _GKA - 1)
    def _():
        idx_ref[0, :, :] = ridx_ref[...]
        s = jnp.sum(rmin_ref[...])

        @pl.when(n == 0)
        def _():
            acc_ref[0, 0] = s

        @pl.when(n > 0)
        def _():
            acc_ref[0, 0] += s

        @pl.when(n == _GNA - 1)
        def _():
            loss_ref[0, 0] = acc_ref[0, 0] * ((1.0 + _BETA) / (_N * _D))


def _distances_argmin(z_flat, codebook):
    return pl.pallas_call(
        _dist_body,
        grid=(_GNA, _GKA),
        in_specs=[
            pl.BlockSpec((_TNA, _D), lambda n, k: (n, 0)),
            pl.BlockSpec((_TKA, _D), lambda n, k: (k, 0)),
        ],
        out_specs=[
            pl.BlockSpec((_TNA, _TKA), lambda n, k: (n, k)),
            pl.BlockSpec((1, _TNA, 1), lambda n, k: (n, 0, 0)),
            pl.BlockSpec(memory_space=pltpu.SMEM, block_shape=(1, 1),
                         index_map=lambda n, k: (0, 0)),
        ],
        out_shape=[
            jax.ShapeDtypeStruct((_N, _K), jnp.float32),
            jax.ShapeDtypeStruct((_GNA, _TNA, 1), jnp.int32),
            jax.ShapeDtypeStruct((1, 1), jnp.float32),
        ],
        scratch_shapes=[
            pltpu.VMEM((_TNA, 1), jnp.float32),
            pltpu.VMEM((_TNA, 1), jnp.int32),
            pltpu.SMEM((1, 1), jnp.float32),
            pltpu.VMEM((_TNA, 1), jnp.float32),
        ],
    )(z_flat, codebook)


def _onehot_body(idx_ref, enc_ref, perp_ref, counts_ref):
    n = pl.program_id(0)
    k = pl.program_id(1)
    idxv = idx_ref[0, :, :]
    col = lax.broadcasted_iota(jnp.int32, (1, _TKB), 1) + k * _TKB
    enc = jnp.where(col == idxv, 1.0, 0.0)
    enc_ref[...] = enc
    cs = jnp.sum(enc, axis=0, keepdims=True)

    @pl.when(n == 0)
    def _():
        counts_ref[pl.ds(k, 1), :] = cs

    @pl.when(n > 0)
    def _():
        counts_ref[pl.ds(k, 1), :] += cs

    @pl.when((n == _GNB - 1) & (k == _GKB - 1))
    def _():
        p = counts_ref[...] * (1.0 / _N)
        ent = jnp.sum(p * jnp.log(p + 1e-10))
        perp_ref[0, 0] = jnp.exp(-ent)


def _onehot_perplexity(idx3):
    return pl.pallas_call(
        _onehot_body,
        grid=(_GNB, _GKB),
        in_specs=[
            pl.BlockSpec((1, _TNB, 1), lambda n, k: (n, 0, 0)),
        ],
        out_specs=[
            pl.BlockSpec((_TNB, _TKB), lambda n, k: (n, k)),
            pl.BlockSpec(memory_space=pltpu.SMEM, block_shape=(1, 1),
                         index_map=lambda n, k: (0, 0)),
        ],
        out_shape=[
            jax.ShapeDtypeStruct((_N, _K), jnp.float32),
            jax.ShapeDtypeStruct((1, 1), jnp.float32),
        ],
        scratch_shapes=[
            pltpu.VMEM((_GKB, _TKB), jnp.float32),
        ],
    )(idx3)


def _sc_gather(codebook, idx_flat):
    info = plsc.get_sparse_core_info()
    nc, ns = info.num_cores, info.num_subcores
    nw = nc * ns
    b_per_w = _N // nw
    mesh = plsc.VectorSubcoreMesh(core_axis_name="c", subcore_axis_name="s")

    half = b_per_w // 2

    @functools.partial(
        pl.kernel, mesh=mesh,
        out_type=jax.ShapeDtypeStruct((_N, _D), jnp.float32),
        scratch_types=[
            pltpu.VMEM((half,), jnp.int32),
            pltpu.VMEM((half,), jnp.int32),
            pltpu.VMEM((half, _D), jnp.float32),
            pltpu.VMEM((half, _D), jnp.float32),
            pltpu.SemaphoreType.DMA,
            pltpu.SemaphoreType.DMA,
        ],
    )
    def gather_kernel(cb_hbm, idx_hbm, out_hbm, idx0, idx1, rows0, rows1,
                      sem0, sem1):
        wid = lax.axis_index("s") * nc + lax.axis_index("c")
        base = wid * b_per_w
        pltpu.sync_copy(idx_hbm.at[pl.ds(base, half)], idx0)
        cp0 = pltpu.async_copy(cb_hbm.at[idx0], rows0, sem0)
        pltpu.sync_copy(idx_hbm.at[pl.ds(base + half, half)], idx1)
        cp1 = pltpu.async_copy(cb_hbm.at[idx1], rows1, sem1)
        cp0.wait()
        pltpu.sync_copy(rows0, out_hbm.at[pl.ds(base, half)])
        cp1.wait()
        pltpu.sync_copy(rows1, out_hbm.at[pl.ds(base + half, half)])

    return gather_kernel(codebook, idx_flat)


def kernel(z, codebook):
    B, D, H, W = z.shape
    z_flat = jnp.transpose(z, (0, 2, 3, 1)).reshape(_N, _D)
    distances, idx3, loss = _distances_argmin(z_flat, codebook)
    encoding_indices = idx3.reshape(_N)
    encodings, perp = _onehot_perplexity(idx3.reshape(_GNB, _TNB, 1))
    zq_flat = _sc_gather(codebook, encoding_indices)
    z_quantized = jnp.transpose(zq_flat.reshape(B, H, W, D), (0, 3, 1, 2))
    loss = loss.reshape(())
    perplexity = perp.reshape(())
    return (z_quantized, loss, perplexity, encodings, encoding_indices,
            distances)


# speedup vs baseline: 1.0787x; 1.0023x over previous
import functools

import jax
import jax.numpy as jnp
from jax import lax
from jax.experimental import pallas as pl
from jax.experimental.pallas import tpu as pltpu
from jax.experimental.pallas import tpu_sc as plsc

_BETA = 0.25

_N = 8192
_K = 8192
_D = 256

_TNA = 2048
_TKA = 2048
_GNA = _N // _TNA
_GKA = _K // _TKA
_TNB = 1024
_TKB = 4096
_GNB = _N // _TNB
_GKB = _K // _TKB


def _dist_body(z_ref, cb_ref, dist_ref, idx_ref, loss_ref, rmin_ref, ridx_ref,
               acc_ref, zn_ref, z2_ref):
    n = pl.program_id(0)
    k = pl.program_id(1)
    zt = z_ref[...]
    ct = cb_ref[...]

    @pl.when(k == 0)
    def _():
        zn_ref[...] = jnp.sum(zt * zt, axis=1, keepdims=True)
        z2_ref[...] = zt * (-2.0)

    zn = zn_ref[...]
    cn = jnp.sum(ct * ct, axis=1)
    mm = lax.dot_general(z2_ref[...], ct, (((1,), (1,)), ((), ())),
                         preferred_element_type=jnp.float32)
    dist = zn + cn[None, :] + mm
    dist_ref[...] = dist

    lmin = jnp.min(dist, axis=1, keepdims=True)
    iota = lax.broadcasted_iota(jnp.int32, (1, _TKA), 1).astype(jnp.float32)
    cand = jnp.min(jnp.where(dist == lmin, iota, jnp.float32(_TKA)),
                   axis=1, keepdims=True)
    larg = cand.astype(jnp.int32) + k * _TKA

    @pl.when(k == 0)
    def _():
        rmin_ref[...] = lmin
        ridx_ref[...] = larg

    @pl.when(k > 0)
    def _():
        pm = rmin_ref[...]
        upd = lmin < pm
        rmin_ref[...] = jnp.where(upd, lmin, pm)
        ridx_ref[...] = jnp.where(upd, larg, ridx_ref[...])

    @pl.when(k == _GKA - 1)
    def _():
        idx_ref[0, :, :] = ridx_ref[...]
        s = jnp.sum(rmin_ref[...])

        @pl.when(n == 0)
        def _():
            acc_ref[0, 0] = s

        @pl.when(n > 0)
        def _():
            acc_ref[0, 0] += s

        @pl.when(n == _GNA - 1)
        def _():
            loss_ref[0, 0] = acc_ref[0, 0] * ((1.0 + _BETA) / (_N * _D))


def _distances_argmin(z_flat, codebook):
    return pl.pallas_call(
        _dist_body,
        grid=(_GNA, _GKA),
        in_specs=[
            pl.BlockSpec((_TNA, _D), lambda n, k: (n, 0)),
            pl.BlockSpec((_TKA, _D), lambda n, k: (k, 0)),
        ],
        out_specs=[
            pl.BlockSpec((_TNA, _TKA), lambda n, k: (n, k)),
            pl.BlockSpec((1, _TNA, 1), lambda n, k: (n, 0, 0)),
            pl.BlockSpec(memory_space=pltpu.SMEM, block_shape=(1, 1),
                         index_map=lambda n, k: (0, 0)),
        ],
        out_shape=[
            jax.ShapeDtypeStruct((_N, _K), jnp.float32),
            jax.ShapeDtypeStruct((_GNA, _TNA, 1), jnp.int32),
            jax.ShapeDtypeStruct((1, 1), jnp.float32),
        ],
        scratch_shapes=[
            pltpu.VMEM((_TNA, 1), jnp.float32),
            pltpu.VMEM((_TNA, 1), jnp.int32),
            pltpu.SMEM((1, 1), jnp.float32),
            pltpu.VMEM((_TNA, 1), jnp.float32),
            pltpu.VMEM((_TNA, _D), jnp.float32),
        ],
    )(z_flat, codebook)


def _onehot_body(idx_ref, enc_ref, perp_ref, counts_ref):
    n = pl.program_id(0)
    k = pl.program_id(1)
    idxv = idx_ref[0, :, :]
    col = lax.broadcasted_iota(jnp.int32, (1, _TKB), 1) + k * _TKB
    enc = jnp.where(col == idxv, 1.0, 0.0)
    enc_ref[...] = enc
    cs = jnp.sum(enc, axis=0, keepdims=True)

    @pl.when(n == 0)
    def _():
        counts_ref[pl.ds(k, 1), :] = cs

    @pl.when(n > 0)
    def _():
        counts_ref[pl.ds(k, 1), :] += cs

    @pl.when((n == _GNB - 1) & (k == _GKB - 1))
    def _():
        p = counts_ref[...] * (1.0 / _N)
        ent = jnp.sum(p * jnp.log(p + 1e-10))
        perp_ref[0, 0] = jnp.exp(-ent)


def _onehot_perplexity(idx3):
    return pl.pallas_call(
        _onehot_body,
        grid=(_GNB, _GKB),
        in_specs=[
            pl.BlockSpec((1, _TNB, 1), lambda n, k: (n, 0, 0)),
        ],
        out_specs=[
            pl.BlockSpec((_TNB, _TKB), lambda n, k: (n, k)),
            pl.BlockSpec(memory_space=pltpu.SMEM, block_shape=(1, 1),
                         index_map=lambda n, k: (0, 0)),
        ],
        out_shape=[
            jax.ShapeDtypeStruct((_N, _K), jnp.float32),
            jax.ShapeDtypeStruct((1, 1), jnp.float32),
        ],
        scratch_shapes=[
            pltpu.VMEM((_GKB, _TKB), jnp.float32),
        ],
    )(idx3)


def _sc_gather(codebook, idx_flat):
    info = plsc.get_sparse_core_info()
    nc, ns = info.num_cores, info.num_subcores
    nw = nc * ns
    b_per_w = _N // nw
    mesh = plsc.VectorSubcoreMesh(core_axis_name="c", subcore_axis_name="s")

    half = b_per_w // 2

    @functools.partial(
        pl.kernel, mesh=mesh,
        out_type=jax.ShapeDtypeStruct((_N, _D), jnp.float32),
        scratch_types=[
            pltpu.VMEM((half,), jnp.int32),
            pltpu.VMEM((half,), jnp.int32),
            pltpu.VMEM((half, _D), jnp.float32),
            pltpu.VMEM((half, _D), jnp.float32),
            pltpu.SemaphoreType.DMA,
            pltpu.SemaphoreType.DMA,
        ],
    )
    def gather_kernel(cb_hbm, idx_hbm, out_hbm, idx0, idx1, rows0, rows1,
                      sem0, sem1):
        wid = lax.axis_index("s") * nc + lax.axis_index("c")
        base = wid * b_per_w
        pltpu.sync_copy(idx_hbm.at[pl.ds(base, half)], idx0)
        cp0 = pltpu.async_copy(cb_hbm.at[idx0], rows0, sem0)
        pltpu.sync_copy(idx_hbm.at[pl.ds(base + half, half)], idx1)
        cp1 = pltpu.async_copy(cb_hbm.at[idx1], rows1, sem1)
        cp0.wait()
        pltpu.sync_copy(rows0, out_hbm.at[pl.ds(base, half)])
        cp1.wait()
        pltpu.sync_copy(rows1, out_hbm.at[pl.ds(base + half, half)])

    return gather_kernel(codebook, idx_flat)


def kernel(z, codebook):
    B, D, H, W = z.shape
    z_flat = jnp.transpose(z, (0, 2, 3, 1)).reshape(_N, _D)
    distances, idx3, loss = _distances_argmin(z_flat, codebook)
    encoding_indices = idx3.reshape(_N)
    encodings, perp = _onehot_perplexity(idx3.reshape(_GNB, _TNB, 1))
    zq_flat = _sc_gather(codebook, encoding_indices)
    z_quantized = jnp.transpose(zq_flat.reshape(B, H, W, D), (0, 3, 1, 2))
    loss = loss.reshape(())
    perplexity = perp.reshape(())
    return (z_quantized, loss, perplexity, encodings, encoding_indices,
            distances)
